# baseline (speedup 1.0000x reference)
import functools

import jax
import jax.numpy as jnp
from jax.experimental import pallas as pl
from jax.experimental.pallas import tpu as pltpu

IN_DIM = 1024
HIDDEN = 2048
N_EXPERTS = 8
BT = 256
NBLK = (4096 // BT) + N_EXPERTS - 1
NTOK = 4096


def _router_body(x_ref, wsw_ref, bsw_ref, routes_ref, pmax_ref):
    x = x_ref[...]
    w = wsw_ref[...]
    logits = jax.lax.dot_general(
        x, w, (((1,), (1,)), ((), ())),
        preferred_element_type=jnp.float32) + bsw_ref[...]
    m = jnp.max(logits, axis=1, keepdims=True)
    s = jnp.sum(jnp.exp(logits - m), axis=1, keepdims=True)
    iota = jax.lax.broadcasted_iota(jnp.int32, logits.shape, 1)
    cand = jnp.where(logits == m, iota, N_EXPERTS)
    routes_ref[...] = jnp.min(cand, axis=1, keepdims=True)
    pmax_ref[...] = 1.0 / s


def _ffn_body(eid_ref, xs_ref, w1_ref, b1_ref, w2_ref, b2_ref, pmax_ref,
              out_ref):
    del eid_ref
    x = xs_ref[...]
    h = jax.lax.dot_general(
        x, w1_ref[0], (((1,), (1,)), ((), ())),
        preferred_element_type=jnp.float32) + b1_ref[0]
    h = jnp.maximum(h, 0.0)
    o = jax.lax.dot_general(
        h, w2_ref[0], (((1,), (1,)), ((), ())),
        preferred_element_type=jnp.float32) + b2_ref[0]
    out_ref[...] = o * pmax_ref[...]


def kernel(x, W_sw, b_sw, W1, b1, W2, b2):
    B, N, T, C = x.shape
    xf = x.reshape(-1, C)

    routes2, pmax2 = pl.pallas_call(
        _router_body,
        grid=(NTOK // BT,),
        in_specs=[
            pl.BlockSpec((BT, C), lambda i: (i, 0)),
            pl.BlockSpec((N_EXPERTS, C), lambda i: (0, 0)),
            pl.BlockSpec((1, N_EXPERTS), lambda i: (0, 0)),
        ],
        out_specs=[
            pl.BlockSpec((BT, 1), lambda i: (i, 0)),
            pl.BlockSpec((BT, 1), lambda i: (i, 0)),
        ],
        out_shape=[
            jax.ShapeDtypeStruct((NTOK, 1), jnp.int32),
            jax.ShapeDtypeStruct((NTOK, 1), jnp.float32),
        ],
    )(xf, W_sw, b_sw.reshape(1, N_EXPERTS))
    routes = routes2[:, 0]
    pmax = pmax2[:, 0]

    sort_idx = jnp.argsort(routes).astype(jnp.int32)
    counts = jnp.bincount(routes, length=N_EXPERTS).astype(jnp.int32)
    offsets = jnp.concatenate(
        [jnp.zeros((1,), jnp.int32), jnp.cumsum(counts).astype(jnp.int32)])
    blocks_e = (counts + BT - 1) // BT
    cum_blocks = jnp.cumsum(blocks_e).astype(jnp.int32)
    pad_off = cum_blocks - blocks_e
    k_blocks = cum_blocks[-1]

    bid = jnp.arange(NBLK, dtype=jnp.int32)
    eid_raw = jnp.searchsorted(cum_blocks, bid, side='right').astype(jnp.int32)
    last_eid = eid_raw[jnp.maximum(k_blocks - 1, 0)]
    eids = jnp.where(bid < k_blocks, jnp.minimum(eid_raw, N_EXPERTS - 1),
                     last_eid)

    j = jnp.arange(NTOK, dtype=jnp.int32)
    e_of_j = jnp.searchsorted(offsets[1:], j, side='right').astype(jnp.int32)
    pos = pad_off[e_of_j] * BT + (j - offsets[e_of_j])
    src = jnp.zeros((NBLK * BT,), jnp.int32).at[pos].set(sort_idx)
    gpos = jnp.zeros((NTOK,), jnp.int32).at[sort_idx].set(pos)

    xs = jnp.take(xf, src, axis=0)
    pmax_p = jnp.take(pmax, src, axis=0)[:, None]

    out_p = pl.pallas_call(
        _ffn_body,
        grid_spec=pltpu.PrefetchScalarGridSpec(
            num_scalar_prefetch=1,
            grid=(NBLK,),
            in_specs=[
                pl.BlockSpec((BT, C), lambda i, eid: (i, 0)),
                pl.BlockSpec((1, HIDDEN, C), lambda i, eid: (eid[i], 0, 0)),
                pl.BlockSpec((1, 1, HIDDEN), lambda i, eid: (eid[i], 0, 0)),
                pl.BlockSpec((1, C, HIDDEN), lambda i, eid: (eid[i], 0, 0)),
                pl.BlockSpec((1, 1, C), lambda i, eid: (eid[i], 0, 0)),
                pl.BlockSpec((BT, 1), lambda i, eid: (i, 0)),
            ],
            out_specs=pl.BlockSpec((BT, C), lambda i, eid: (i, 0)),
        ),
        out_shape=jax.ShapeDtypeStruct((NBLK * BT, C), jnp.float32),
    )(eids, xs, W1, b1.reshape(N_EXPERTS, 1, HIDDEN), W2,
      b2.reshape(N_EXPERTS, 1, C), pmax_p)

    out = jnp.take(out_p, gpos, axis=0)
    return out.reshape(B, N, T, C)


# speedup vs baseline: 1.3465x; 1.3465x over previous
import functools

import jax
import jax.numpy as jnp
from jax import lax
from jax.experimental import pallas as pl
from jax.experimental.pallas import tpu as pltpu
from jax.experimental.pallas import tpu_sc as plsc

IN_DIM = 1024
HIDDEN = 2048
N_EXPERTS = 8
BT = 256
NBLK = (4096 // BT) + N_EXPERTS - 1
NTOK = 4096


def _make_sc_row_gather(V, D, B):
    info = plsc.get_sparse_core_info()
    NC, NS = info.num_cores, info.num_subcores
    NW = NC * NS
    CH = 8
    R = 4
    assert B % (CH * NW) == 0
    b_per_w = B // NW
    n_chunks = b_per_w // CH
    mesh = plsc.VectorSubcoreMesh(core_axis_name="c", subcore_axis_name="s")

    @functools.partial(
        pl.kernel, mesh=mesh,
        out_type=jax.ShapeDtypeStruct((B, D), jnp.float32),
        scratch_types=[
            pltpu.VMEM((b_per_w,), jnp.int32),
            pltpu.VMEM((R, CH, D), jnp.float32),
        ] + [pltpu.SemaphoreType.DMA] * R,
    )
    def gather_k(table_hbm, idx_hbm, out_hbm, idx_v, rows_v, *sems):
        wid = lax.axis_index("s") * NC + lax.axis_index("c")
        base = wid * b_per_w
        pltpu.sync_copy(idx_hbm.at[pl.ds(base, b_per_w)], idx_v)

        def start(c, r):
            pltpu.async_copy(
                table_hbm.at[idx_v.at[pl.ds(c * CH, CH)]],
                rows_v.at[r], sems[r])

        for r in range(min(R, n_chunks)):
            start(r, r)
        for c in range(n_chunks):
            r = c % R
            pltpu.make_async_copy(
                table_hbm.at[idx_v.at[pl.ds(c * CH, CH)]],
                rows_v.at[r], sems[r]).wait()
            pltpu.sync_copy(rows_v.at[r],
                            out_hbm.at[pl.ds(base + c * CH, CH)])
            if c + R < n_chunks:
                start(c + R, r)

    return gather_k


def _router_body(x_ref, wsw_ref, bsw_ref, routes_ref, pmax_ref):
    x = x_ref[...]
    w = wsw_ref[...]
    logits = jax.lax.dot_general(
        x, w, (((1,), (1,)), ((), ())),
        preferred_element_type=jnp.float32) + bsw_ref[...]
    m = jnp.max(logits, axis=1, keepdims=True)
    s = jnp.sum(jnp.exp(logits - m), axis=1, keepdims=True)
    iota = jax.lax.broadcasted_iota(jnp.int32, logits.shape, 1)
    cand = jnp.where(logits == m, iota, N_EXPERTS)
    routes_ref[...] = jnp.min(cand, axis=1, keepdims=True)
    pmax_ref[...] = 1.0 / s


def _ffn_body(eid_ref, xs_ref, w1_ref, b1_ref, w2_ref, b2_ref, pmax_ref,
              out_ref):
    del eid_ref
    x = xs_ref[...]
    h = jax.lax.dot_general(
        x, w1_ref[0], (((1,), (1,)), ((), ())),
        preferred_element_type=jnp.float32) + b1_ref[0]
    h = jnp.maximum(h, 0.0)
    o = jax.lax.dot_general(
        h, w2_ref[0], (((1,), (1,)), ((), ())),
        preferred_element_type=jnp.float32) + b2_ref[0]
    out_ref[...] = o * pmax_ref[...]


def kernel(x, W_sw, b_sw, W1, b1, W2, b2):
    B, N, T, C = x.shape
    xf = x.reshape(-1, C)

    routes2, pmax2 = pl.pallas_call(
        _router_body,
        grid=(NTOK // BT,),
        in_specs=[
            pl.BlockSpec((BT, C), lambda i: (i, 0)),
            pl.BlockSpec((N_EXPERTS, C), lambda i: (0, 0)),
            pl.BlockSpec((1, N_EXPERTS), lambda i: (0, 0)),
        ],
        out_specs=[
            pl.BlockSpec((BT, 1), lambda i: (i, 0)),
            pl.BlockSpec((BT, 1), lambda i: (i, 0)),
        ],
        out_shape=[
            jax.ShapeDtypeStruct((NTOK, 1), jnp.int32),
            jax.ShapeDtypeStruct((NTOK, 1), jnp.float32),
        ],
    )(xf, W_sw, b_sw.reshape(1, N_EXPERTS))
    routes = routes2[:, 0]
    pmax = pmax2[:, 0]

    sort_idx = jnp.argsort(routes).astype(jnp.int32)
    counts = jnp.bincount(routes, length=N_EXPERTS).astype(jnp.int32)
    offsets = jnp.concatenate(
        [jnp.zeros((1,), jnp.int32), jnp.cumsum(counts).astype(jnp.int32)])
    blocks_e = (counts + BT - 1) // BT
    cum_blocks = jnp.cumsum(blocks_e).astype(jnp.int32)
    pad_off = cum_blocks - blocks_e
    k_blocks = cum_blocks[-1]

    bid = jnp.arange(NBLK, dtype=jnp.int32)
    eid_raw = jnp.searchsorted(cum_blocks, bid, side='right').astype(jnp.int32)
    last_eid = eid_raw[jnp.maximum(k_blocks - 1, 0)]
    eids = jnp.where(bid < k_blocks, jnp.minimum(eid_raw, N_EXPERTS - 1),
                     last_eid)

    j = jnp.arange(NTOK, dtype=jnp.int32)
    e_of_j = jnp.searchsorted(offsets[1:], j, side='right').astype(jnp.int32)
    pos = pad_off[e_of_j] * BT + (j - offsets[e_of_j])
    src = jnp.zeros((NBLK * BT,), jnp.int32).at[pos].set(sort_idx)
    gpos = jnp.zeros((NTOK,), jnp.int32).at[sort_idx].set(pos)

    xs = _make_sc_row_gather(NTOK, C, NBLK * BT)(xf, src)
    pmax_p = jnp.take(pmax, src, axis=0)[:, None]

    out_p = pl.pallas_call(
        _ffn_body,
        grid_spec=pltpu.PrefetchScalarGridSpec(
            num_scalar_prefetch=1,
            grid=(NBLK,),
            in_specs=[
                pl.BlockSpec((BT, C), lambda i, eid: (i, 0)),
                pl.BlockSpec((1, HIDDEN, C), lambda i, eid: (eid[i], 0, 0)),
                pl.BlockSpec((1, 1, HIDDEN), lambda i, eid: (eid[i], 0, 0)),
                pl.BlockSpec((1, C, HIDDEN), lambda i, eid: (eid[i], 0, 0)),
                pl.BlockSpec((1, 1, C), lambda i, eid: (eid[i], 0, 0)),
                pl.BlockSpec((BT, 1), lambda i, eid: (i, 0)),
            ],
            out_specs=pl.BlockSpec((BT, C), lambda i, eid: (i, 0)),
        ),
        out_shape=jax.ShapeDtypeStruct((NBLK * BT, C), jnp.float32),
    )(eids, xs, W1, b1.reshape(N_EXPERTS, 1, HIDDEN), W2,
      b2.reshape(N_EXPERTS, 1, C), pmax_p)

    out = _make_sc_row_gather(NBLK * BT, C, NTOK)(out_p, gpos)
    return out.reshape(B, N, T, C)


# speedup vs baseline: 1.5889x; 1.1799x over previous
import functools

import jax
import jax.numpy as jnp
from jax import lax
from jax.experimental import pallas as pl
from jax.experimental.pallas import tpu as pltpu
from jax.experimental.pallas import tpu_sc as plsc

IN_DIM = 1024
HIDDEN = 2048
N_EXPERTS = 8
BT = 256
NBLK = (4096 // BT) + N_EXPERTS - 1
NTOK = 4096
NPAD = (NBLK + 1) * BT


def _make_sc_row_gather(V, D, B):
    info = plsc.get_sparse_core_info()
    NC, NS = info.num_cores, info.num_subcores
    NW = NC * NS
    CH = 32
    R = 3
    assert B % (CH * NW) == 0
    b_per_w = B // NW
    n_chunks = b_per_w // CH
    mesh = plsc.VectorSubcoreMesh(core_axis_name="c", subcore_axis_name="s")

    @functools.partial(
        pl.kernel, mesh=mesh,
        out_type=jax.ShapeDtypeStruct((B, D), jnp.float32),
        scratch_types=[
            pltpu.VMEM((b_per_w,), jnp.int32),
            pltpu.VMEM((R, CH, D), jnp.float32),
        ] + [pltpu.SemaphoreType.DMA] * (2 * R),
    )
    def gather_k(table_hbm, idx_hbm, out_hbm, idx_v, rows_v, *sems):
        g_sems, o_sems = sems[:R], sems[R:]
        wid = lax.axis_index("s") * NC + lax.axis_index("c")
        base = wid * b_per_w
        pltpu.sync_copy(idx_hbm.at[pl.ds(base, b_per_w)], idx_v)

        def g_copy(c):
            return pltpu.make_async_copy(
                table_hbm.at[idx_v.at[pl.ds(c * CH, CH)]],
                rows_v.at[c % R], g_sems[c % R])

        def o_copy(c):
            return pltpu.make_async_copy(
                rows_v.at[c % R],
                out_hbm.at[pl.ds(base + c * CH, CH)], o_sems[c % R])

        for c in range(min(R, n_chunks)):
            g_copy(c).start()
        for c in range(n_chunks):
            g_copy(c).wait()
            o_copy(c).start()
            if c + R < n_chunks:
                o_copy(c).wait()
                g_copy(c + R).start()
        for c in range(max(n_chunks - R, 0), n_chunks):
            o_copy(c).wait()

    return gather_k


def _router_body(x_ref, wsw_ref, bsw_ref, routes_ref):
    x = x_ref[...]
    w = wsw_ref[...]
    logits = jax.lax.dot_general(
        x, w, (((1,), (1,)), ((), ())),
        preferred_element_type=jnp.float32) + bsw_ref[...]
    m = jnp.max(logits, axis=1, keepdims=True)
    iota = jax.lax.broadcasted_iota(jnp.int32, logits.shape, 1)
    cand = jnp.where(logits == m, iota, N_EXPERTS)
    routes_ref[...] = jnp.min(cand, axis=1, keepdims=True)


def _ffn_body(eid_ref, xs_ref, w1_ref, b1_ref, w2_ref, b2_ref, wsw_ref,
              bsw_ref, out_ref):
    del eid_ref
    x = xs_ref[...]
    logits = jax.lax.dot_general(
        x, wsw_ref[...], (((1,), (1,)), ((), ())),
        preferred_element_type=jnp.float32) + bsw_ref[...]
    m = jnp.max(logits, axis=1, keepdims=True)
    scale = 1.0 / jnp.sum(jnp.exp(logits - m), axis=1, keepdims=True)

    h = jax.lax.dot_general(
        x, w1_ref[0], (((1,), (1,)), ((), ())),
        preferred_element_type=jnp.float32) + b1_ref[0]
    h = jnp.maximum(h, 0.0)
    o = jax.lax.dot_general(
        h, w2_ref[0], (((1,), (1,)), ((), ())),
        preferred_element_type=jnp.float32) + b2_ref[0]
    out_ref[...] = o * scale


def kernel(x, W_sw, b_sw, W1, b1, W2, b2):
    B, N, T, C = x.shape
    xf = x.reshape(-1, C)
    bsw2 = b_sw.reshape(1, N_EXPERTS)

    routes2 = pl.pallas_call(
        _router_body,
        grid=(NTOK // BT,),
        in_specs=[
            pl.BlockSpec((BT, C), lambda i: (i, 0)),
            pl.BlockSpec((N_EXPERTS, C), lambda i: (0, 0)),
            pl.BlockSpec((1, N_EXPERTS), lambda i: (0, 0)),
        ],
        out_specs=pl.BlockSpec((BT, 1), lambda i: (i, 0)),
        out_shape=jax.ShapeDtypeStruct((NTOK, 1), jnp.int32),
    )(xf, W_sw, bsw2)
    routes = routes2[:, 0]

    onehot = (routes[:, None] == jnp.arange(N_EXPERTS, dtype=jnp.int32)
              ).astype(jnp.int32)
    csum = jnp.cumsum(onehot, axis=0)
    counts = csum[-1]
    rank = jnp.sum(onehot * (csum - 1), axis=1)
    blocks_e = (counts + BT - 1) // BT
    cum_blocks = jnp.cumsum(blocks_e)
    pad_off = cum_blocks - blocks_e
    k_blocks = cum_blocks[-1]

    bid = jnp.arange(NBLK, dtype=jnp.int32)
    eid_raw = jnp.sum((cum_blocks[None, :] <= bid[:, None]).astype(jnp.int32),
                      axis=1)
    last_eid = jnp.sum((cum_blocks <= (k_blocks - 1)).astype(jnp.int32))
    eids = jnp.where(bid < k_blocks, jnp.minimum(eid_raw, N_EXPERTS - 1),
                     last_eid).astype(jnp.int32)

    pos = (jnp.sum(onehot * pad_off[None, :], axis=1) * BT + rank
           ).astype(jnp.int32)
    src = jnp.zeros((NPAD,), jnp.int32).at[pos].set(
        jnp.arange(NTOK, dtype=jnp.int32))

    xs = _make_sc_row_gather(NTOK, C, NPAD)(xf, src)

    out_p = pl.pallas_call(
        _ffn_body,
        grid_spec=pltpu.PrefetchScalarGridSpec(
            num_scalar_prefetch=1,
            grid=(NBLK,),
            in_specs=[
                pl.BlockSpec((BT, C), lambda i, eid: (i, 0)),
                pl.BlockSpec((1, HIDDEN, C), lambda i, eid: (eid[i], 0, 0)),
                pl.BlockSpec((1, 1, HIDDEN), lambda i, eid: (eid[i], 0, 0)),
                pl.BlockSpec((1, C, HIDDEN), lambda i, eid: (eid[i], 0, 0)),
                pl.BlockSpec((1, 1, C), lambda i, eid: (eid[i], 0, 0)),
                pl.BlockSpec((N_EXPERTS, C), lambda i, eid: (0, 0)),
                pl.BlockSpec((1, N_EXPERTS), lambda i, eid: (0, 0)),
            ],
            out_specs=pl.BlockSpec((BT, C), lambda i, eid: (i, 0)),
        ),
        out_shape=jax.ShapeDtypeStruct((NBLK * BT, C), jnp.float32),
    )(eids, xs, W1, b1.reshape(N_EXPERTS, 1, HIDDEN), W2,
      b2.reshape(N_EXPERTS, 1, C), W_sw, bsw2)

    out = _make_sc_row_gather(NBLK * BT, C, NTOK)(out_p, pos)
    return out.reshape(B, N, T, C)
